# baseline (speedup 1.0000x reference)
import jax
import jax.numpy as jnp
from jax.experimental import pallas as pl

_SRC = 50
_V = 64
_B = 16384
_F = 100
_OUT_F = _F - 1 + _V

_ROWS_PER_BLOCK = 1024


def _body(x_ref, eps_ref, o_ref):
    x = x_ref[...]
    eps = eps_ref[0, 0]
    col = x[:, _SRC:_SRC + 1]
    iota = jax.lax.broadcasted_iota(
        jnp.int32, (x.shape[0], _V), 1).astype(jnp.float32)
    onehot = eps * (col == iota).astype(jnp.float32)
    o_ref[...] = jnp.concatenate(
        [x[:, :_SRC], onehot, x[:, _SRC + 1:]], axis=1)


def kernel(x, eps):
    eps2 = jnp.reshape(eps, (1, 1))
    grid = (_B // _ROWS_PER_BLOCK,)
    return pl.pallas_call(
        _body,
        grid=grid,
        in_specs=[
            pl.BlockSpec((_ROWS_PER_BLOCK, _F), lambda i: (i, 0)),
            pl.BlockSpec((1, 1), lambda i: (0, 0)),
        ],
        out_specs=pl.BlockSpec((_ROWS_PER_BLOCK, _OUT_F), lambda i: (i, 0)),
        out_shape=jax.ShapeDtypeStruct((_B, _OUT_F), jnp.float32),
    )(x, eps2)
